# Initial kernel scaffold; baseline (speedup 1.0000x reference)
import jax
import jax.numpy as jnp
from jax.experimental import pallas as pl


def kernel(x, edge_index, edge_weight, W1, b1, g1, be1, W2, b2, g2, be2, W3, b3):
    raise NotImplementedError("write your pallas kernel here")


# speedup vs baseline: 8.5178x; 8.5178x over previous
import functools

import jax
import jax.numpy as jnp
from jax import lax
from jax.experimental import pallas as pl
from jax.experimental.pallas import tpu as pltpu
from jax.experimental.pallas import tpu_sc as plsc

N = 10000
E = 640000
NC = 2
NS = 16
NW = NC * NS
EPW = E // NW
K = 80
ITERS = EPW // K
NROWS = 1000
NSPLIT = N // NROWS

_mesh = plsc.VectorSubcoreMesh(core_axis_name="c", subcore_axis_name="s")


def _deg_body(dst_hbm, ew_hbm, pdeg0_hbm, pdeg1_hbm, idx_v, val_v, zbuf,
              deg_sh):
    c = lax.axis_index("c")
    s = lax.axis_index("s")
    w = c * NS + s

    def zfill(i, cy):
        zbuf[pl.ds(i * 16, 16)] = jnp.zeros((16,), jnp.float32)
        return cy

    lax.fori_loop(0, 64, zfill, 0)

    @pl.when(s < NSPLIT)
    def _():
        pltpu.sync_copy(zbuf.at[pl.ds(0, NROWS)],
                        deg_sh.at[pl.ds(s * NROWS, NROWS)])

    plsc.subcore_barrier()
    g0 = w * EPW

    def body(i, carry):
        off = g0 + i * K
        pltpu.sync_copy(dst_hbm.at[pl.ds(off, K)], idx_v)
        pltpu.sync_copy(ew_hbm.at[pl.ds(off, K)], val_v)
        pltpu.sync_copy(val_v, deg_sh.at[idx_v], add=True)
        return carry

    lax.fori_loop(0, ITERS, body, 0)
    plsc.subcore_barrier()

    @pl.when(s < NSPLIT)
    def _():
        pltpu.sync_copy(deg_sh.at[pl.ds(s * NROWS, NROWS)],
                        zbuf.at[pl.ds(0, NROWS)])

    @pl.when(jnp.logical_and(s < NSPLIT, c == 0))
    def _():
        pltpu.sync_copy(zbuf.at[pl.ds(0, NROWS)],
                        pdeg0_hbm.at[pl.ds(s * NROWS, NROWS)])

    @pl.when(jnp.logical_and(s < NSPLIT, c == 1))
    def _():
        pltpu.sync_copy(zbuf.at[pl.ds(0, NROWS)],
                        pdeg1_hbm.at[pl.ds(s * NROWS, NROWS)])


CHUNK = 200


def _agg_body(D, hs_hbm, src_hbm, dst_hbm, ew_hbm, part_hbm,
              sidx, didx, ew_v, rows, stage, acc_sh):
    c = lax.axis_index("c")
    s = lax.axis_index("s")
    w = c * NS + s

    def zrow(r, cy):
        for ch in range(D // 16):
            stage[r, pl.ds(ch * 16, 16)] = jnp.zeros((16,), jnp.float32)
        return cy

    lax.fori_loop(0, CHUNK, zrow, 0)

    @pl.when(s < NSPLIT)
    def _():
        for j in range(NROWS // CHUNK):
            pltpu.sync_copy(
                stage,
                acc_sh.at[pl.ds(s * NROWS + j * CHUNK, CHUNK)])

    plsc.subcore_barrier()
    g0 = w * EPW

    def body(i, carry):
        off = g0 + i * K
        pltpu.sync_copy(src_hbm.at[pl.ds(off, K)], sidx)
        pltpu.sync_copy(ew_hbm.at[pl.ds(off, K)], ew_v)
        pltpu.sync_copy(dst_hbm.at[pl.ds(off, K)], didx)
        pltpu.sync_copy(hs_hbm.at[sidx], rows)

        def scale(g, cy):
            wv = ew_v[pl.ds(g * 16, 16)]
            base = g * 16
            for j in range(16):
                wk = wv[j]
                for ch in range(D // 16):
                    sl = pl.ds(ch * 16, 16)
                    rows[base + j, sl] = rows[base + j, sl] * wk
            return cy

        lax.fori_loop(0, K // 16, scale, 0)
        pltpu.sync_copy(rows, acc_sh.at[didx], add=True)
        return carry

    lax.fori_loop(0, ITERS, body, 0)
    plsc.subcore_barrier()

    @pl.when(s < NSPLIT)
    def _():
        for j in range(NROWS // CHUNK):
            row0 = s * NROWS + j * CHUNK
            pltpu.sync_copy(acc_sh.at[pl.ds(row0, CHUNK)], stage)
            pltpu.sync_copy(stage, part_hbm.at[c, pl.ds(row0, CHUNK)])


def _make_deg():
    return pl.kernel(
        _deg_body,
        out_type=(jax.ShapeDtypeStruct((N,), jnp.float32),
                  jax.ShapeDtypeStruct((N,), jnp.float32)),
        mesh=_mesh,
        scratch_types=[
            pltpu.VMEM((K,), jnp.int32),
            pltpu.VMEM((K,), jnp.float32),
            pltpu.VMEM((1024,), jnp.float32),
            pltpu.VMEM_SHARED((N,), jnp.float32),
        ],
    )


def _make_agg(D):
    return pl.kernel(
        functools.partial(_agg_body, D),
        out_type=jax.ShapeDtypeStruct((NC, N, D), jnp.float32),
        mesh=_mesh,
        scratch_types=[
            pltpu.VMEM((K,), jnp.int32),
            pltpu.VMEM((K,), jnp.int32),
            pltpu.VMEM((K,), jnp.float32),
            pltpu.VMEM((K, D), jnp.float32),
            pltpu.VMEM((CHUNK, D), jnp.float32),
            pltpu.VMEM_SHARED((N, D), jnp.float32),
        ],
    )


def _prep_body(pdeg_ref, x_ref, w1_ref, dinv_ref, hs_ref):
    deg = pdeg_ref[:, 0:1] + pdeg_ref[:, 1:2] + 1.0
    dinv = lax.rsqrt(deg)
    dinv_ref[...] = dinv
    h = jnp.dot(x_ref[...], w1_ref[...], preferred_element_type=jnp.float32)
    hs_ref[...] = h * dinv


def _mid_body(p_ref, hs_ref, dinv_ref, b_ref, g_ref, be_ref, w_ref, out_ref):
    dinv = dinv_ref[...]
    agg = p_ref[0] + p_ref[1] + hs_ref[...]
    o = agg * dinv + b_ref[...]
    mean = jnp.mean(o, axis=0, keepdims=True)
    var = jnp.mean(o * o, axis=0, keepdims=True) - mean * mean
    y = (o - mean) * lax.rsqrt(var + 1e-5) * g_ref[...] + be_ref[...]
    y = jnp.maximum(y, 0.0)
    h = jnp.dot(y, w_ref[...], preferred_element_type=jnp.float32)
    out_ref[...] = h * dinv


def _mid_noW_body(p_ref, hs_ref, dinv_ref, b_ref, g_ref, be_ref, out_ref):
    dinv = dinv_ref[...]
    agg = p_ref[0] + p_ref[1] + hs_ref[...]
    o = agg * dinv + b_ref[...]
    mean = jnp.mean(o, axis=0, keepdims=True)
    var = jnp.mean(o * o, axis=0, keepdims=True) - mean * mean
    y = (o - mean) * lax.rsqrt(var + 1e-5) * g_ref[...] + be_ref[...]
    y = jnp.maximum(y, 0.0)
    out_ref[...] = y * dinv


def _final_body(p_ref, q_ref, dinv_ref, w3_ref, b_ref, out_ref):
    agg = (p_ref[0] + p_ref[1] + q_ref[...]) * dinv_ref[...]
    out_ref[...] = (
        jnp.dot(agg, w3_ref[...], preferred_element_type=jnp.float32)
        + b_ref[...])


def kernel(x, edge_index, edge_weight, W1, b1, g1, be1, W2, b2, g2, be2, W3, b3):
    src = edge_index[0]
    dst = edge_index[1]
    f32 = jnp.float32

    W3p = jnp.pad(W3, ((0, 0), (0, 16 - W3.shape[1])))
    b3p = jnp.pad(b3, (0, 16 - b3.shape[0]))

    pdeg0, pdeg1 = _make_deg()(dst, edge_weight)
    pdeg_t = jnp.stack([pdeg0, pdeg1], axis=1)

    dinv, hs1 = pl.pallas_call(
        _prep_body,
        out_shape=(jax.ShapeDtypeStruct((N, 1), f32),
                   jax.ShapeDtypeStruct((N, 128), f32)),
    )(pdeg_t, x, W1)

    agg = _make_agg(128)

    p1 = agg(hs1, src, dst, edge_weight)
    hs2 = pl.pallas_call(
        _mid_body,
        out_shape=jax.ShapeDtypeStruct((N, 128), f32),
    )(p1, hs1, dinv, b1.reshape(1, -1), g1.reshape(1, -1), be1.reshape(1, -1), W2)

    p2 = agg(hs2, src, dst, edge_weight)
    q3 = pl.pallas_call(
        _mid_noW_body,
        out_shape=jax.ShapeDtypeStruct((N, 128), f32),
    )(p2, hs2, dinv, b2.reshape(1, -1), g2.reshape(1, -1), be2.reshape(1, -1))

    p3 = agg(q3, src, dst, edge_weight)
    out = pl.pallas_call(
        _final_body,
        out_shape=jax.ShapeDtypeStruct((N, 16), f32),
    )(p3, q3, dinv, W3p, b3p.reshape(1, -1))
    return out[:, :12]
